# baseline (speedup 1.0000x reference)
import functools

import jax
import jax.numpy as jnp
from jax import lax
from jax.experimental import pallas as pl
from jax.experimental.pallas import tpu as pltpu
from jax.experimental.pallas import tpu_sc as plsc

_ALPHA = 4
_FPB = 16


def _fast_body(in_ref, fast_ref):
    fast_ref[...] = in_ref[...]


def _fast_copy(frames):
    C, T, H, W = frames.shape
    return pl.pallas_call(
        _fast_body,
        grid=(T // _FPB,),
        in_specs=[pl.BlockSpec((C, _FPB, H, W), lambda g: (0, g, 0, 0))],
        out_specs=pl.BlockSpec((C, _FPB, H, W), lambda g: (0, g, 0, 0)),
        out_shape=jax.ShapeDtypeStruct((C, T, H, W), frames.dtype),
    )(frames)


def _slow_gather_sc(frames3, T, n_slow):
    NR, H, W = frames3.shape
    C = NR // T
    n_tasks = C * n_slow
    mesh = plsc.VectorSubcoreMesh(core_axis_name="c", subcore_axis_name="s")
    info = plsc.get_sparse_core_info()
    nw = info.num_cores * info.num_subcores

    @functools.partial(
        pl.kernel,
        mesh=mesh,
        out_type=jax.ShapeDtypeStruct((n_tasks, H, W), frames3.dtype),
        scratch_types=[
            pltpu.VMEM((1, H, W), frames3.dtype),
        ],
    )
    def k(frames_hbm, out_hbm, buf):
        wid = lax.axis_index("s") * info.num_cores + lax.axis_index("c")

        def do_task(kk):
            c = kk // n_slow
            j = kk % n_slow
            src = c * T + _ALPHA * j + j // 5
            pltpu.sync_copy(frames_hbm.at[pl.ds(src, 1)], out_hbm.at[pl.ds(kk, 1)])

        do_task(wid)

        @pl.when(wid + nw < n_tasks)
        def _():
            do_task(wid + nw)

    return k(frames3)


def kernel(frames):
    C, T, H, W = frames.shape
    n_slow = T // _ALPHA
    slow3 = _slow_gather_sc(frames.reshape(C * T, H, W), T, n_slow)
    fast = _fast_copy(frames)
    slow = slow3.reshape(C, n_slow, H, W)
    return (slow, fast)


# speedup vs baseline: 7.0175x; 7.0175x over previous
import functools

import jax
import jax.numpy as jnp
from jax import lax
from jax.experimental import pallas as pl
from jax.experimental.pallas import tpu as pltpu
from jax.experimental.pallas import tpu_sc as plsc

_ALPHA = 4
_FPB = 16


def _fast_body(in_ref, fast_ref):
    fast_ref[...] = in_ref[...]


def _fast_copy(frames):
    C, T, H, W = frames.shape
    return pl.pallas_call(
        _fast_body,
        grid=(T // _FPB,),
        in_specs=[pl.BlockSpec((C, _FPB, H, W), lambda g: (0, g, 0, 0))],
        out_specs=pl.BlockSpec((C, _FPB, H, W), lambda g: (0, g, 0, 0)),
        out_shape=jax.ShapeDtypeStruct((C, T, H, W), frames.dtype),
    )(frames)


def _slow_gather_sc(frames3, T, n_slow):
    NR, H, W = frames3.shape
    C = NR // T
    n_tasks = C * n_slow
    mesh = plsc.VectorSubcoreMesh(core_axis_name="c", subcore_axis_name="s")
    info = plsc.get_sparse_core_info()
    nw = info.num_cores * info.num_subcores

    hh = H // 2
    n_half = 2 * n_tasks
    tpw = n_half // nw

    @functools.partial(
        pl.kernel,
        mesh=mesh,
        out_type=jax.ShapeDtypeStruct((n_tasks, H, W), frames3.dtype),
        scratch_types=(
            [pltpu.VMEM((1, hh, W), frames3.dtype) for _ in range(3)]
            + [pltpu.SemaphoreType.DMA for _ in range(4)]
        ),
    )
    def k(frames_hbm, out_hbm, buf0, buf1, buf2, rs0, rs1, rs2, wsem):
        wid = lax.axis_index("s") * info.num_cores + lax.axis_index("c")
        bufs = [buf0, buf1, buf2]
        rsems = [rs0, rs1, rs2]

        reads, dsts = [], []
        for i in range(tpw):
            m = wid + nw * i
            kk = m // 2
            h = (m % 2) * hh
            c = kk // n_slow
            j = kk % n_slow
            src = c * T + _ALPHA * j + j // 5
            rd = pltpu.make_async_copy(
                frames_hbm.at[pl.ds(src, 1), pl.ds(h, hh)], bufs[i], rsems[i]
            )
            rd.start()
            reads.append(rd)
            dsts.append((kk, h))
        writes = []
        for i in range(tpw):
            reads[i].wait()
            kk, h = dsts[i]
            wr = pltpu.make_async_copy(
                bufs[i], out_hbm.at[pl.ds(kk, 1), pl.ds(h, hh)], wsem
            )
            wr.start()
            writes.append(wr)
        for wr in writes:
            wr.wait()

    return k(frames3)


def kernel(frames):
    C, T, H, W = frames.shape
    n_slow = T // _ALPHA
    slow3 = _slow_gather_sc(frames.reshape(C * T, H, W), T, n_slow)
    fast = _fast_copy(frames)
    slow = slow3.reshape(C, n_slow, H, W)
    return (slow, fast)


# speedup vs baseline: 11.1570x; 1.5899x over previous
import jax
import jax.numpy as jnp
from jax.experimental import pallas as pl
from jax.experimental.pallas import tpu as pltpu

_ALPHA = 4
_FPB = 16


def kernel(frames):
    C, T, H, W = frames.shape
    n_slow = T // _ALPHA
    spb = _FPB // _ALPHA
    idx = jnp.linspace(0.0, T - 1, n_slow).astype(jnp.int32)
    off = idx - _ALPHA * jnp.arange(n_slow, dtype=jnp.int32)

    def body(off_ref, in_ref, slow_ref, fast_ref):
        fast_ref[...] = in_ref[...]
        g = pl.program_id(0)
        for s in range(spb):
            o = off_ref[g * spb + s] + s * _ALPHA
            slow_ref[:, pl.ds(s, 1)] = in_ref[:, pl.ds(o, 1)]

    grid_spec = pltpu.PrefetchScalarGridSpec(
        num_scalar_prefetch=1,
        grid=(T // _FPB,),
        in_specs=[
            pl.BlockSpec((C, _FPB, H, W), lambda g, off_r: (0, g, 0, 0)),
        ],
        out_specs=[
            pl.BlockSpec((C, spb, H, W), lambda g, off_r: (0, g, 0, 0)),
            pl.BlockSpec((C, _FPB, H, W), lambda g, off_r: (0, g, 0, 0)),
        ],
    )
    slow, fast = pl.pallas_call(
        body,
        grid_spec=grid_spec,
        out_shape=(
            jax.ShapeDtypeStruct((C, n_slow, H, W), frames.dtype),
            jax.ShapeDtypeStruct((C, T, H, W), frames.dtype),
        ),
        compiler_params=pltpu.CompilerParams(
            dimension_semantics=("parallel",),
        ),
    )(off, frames)
    return (slow, fast)
